# baseline (speedup 1.0000x reference)
import functools

import jax
import jax.numpy as jnp
from jax import lax
from jax.experimental import pallas as pl
from jax.experimental.pallas import tpu as pltpu
from jax.experimental.pallas import tpu_sc as plsc

N = 10000
E = 320000
P = 128

NC = 2
NS = 16
HC = P // NC
EW = E // NS
K = 16
NCHUNK = EW // K
ZR = 80
RPT = 640


NB = 10
LA = 8


def _sc_kernel(e4, w3d, mu_rs, zeros_hbm, agg_out,
               wparts_out, src_v, dst_v, w_v, rows3, sp_acc, sm_acc, acc,
               *sems):
  gsems = sems[:NB]
  ssems = sems[NB:]
  c = lax.axis_index("c")
  s = lax.axis_index("s")

  z16 = jnp.zeros((16,), jnp.float32)

  @pl.when(c == 0)
  def _():
    def zero_scalar(i, _):
      idx = pl.multiple_of(i * 16, 16)
      sp_acc[pl.ds(idx, 16)] = z16
      sm_acc[pl.ds(idx, 16)] = z16
      return 0
    lax.fori_loop(0, N // 16, zero_scalar, 0)

  for t in range(RPT // ZR):
    @pl.when((s < NS - 1) | (t < 5))
    def _():
      r0 = s * RPT + t * ZR
      pltpu.sync_copy(zeros_hbm.at[pl.ds(r0, ZR)],
                      acc.at[pl.ds(r0, ZR)])
  plsc.subcore_barrier()

  pltpu.sync_copy(e4.at[0, s], src_v)
  pltpu.sync_copy(e4.at[1, s], dst_v)

  @pl.when(c == 0)
  def _():
    pltpu.sync_copy(w3d.at[s], w_v)

  def halfsel(jj, _):
    src_v[jj, :] = src_v[jj, :] + c
    return 0

  lax.fori_loop(0, NCHUNK, halfsel, 0)

  def gstart(j, b):
    pltpu.async_copy(mu_rs.at[src_v.at[j]], rows3.at[b], gsems[b])

  def gwait(j, b):
    pltpu.make_async_copy(mu_rs.at[src_v.at[j]], rows3.at[b],
                          gsems[b]).wait()

  def sstart(j, b):
    pltpu.async_copy(rows3.at[b], acc.at[dst_v.at[j]], ssems[b], add=True)

  def swait(j, b):
    pltpu.make_async_copy(rows3.at[b], acc.at[dst_v.at[j]],
                          ssems[b]).wait()

  def scalar_adds(j):
    @pl.when(c == 0)
    def _():
      for m in range(K // 16):
        d16 = dst_v[j, pl.ds(m * 16, 16)]
        w16 = w_v[j, pl.ds(m * 16, 16)]
        plsc.addupdate_scatter(sp_acc, [d16], jnp.maximum(w16, 0.0))
        plsc.addupdate_scatter(sm_acc, [d16], jnp.maximum(-w16, 0.0))

  for u in range(LA):
    gstart(u, u)

  def body(jj, _):
    for u in range(NB):
      j = jj * NB + u
      bn = (u + LA) % NB
      gwait(j, u)
      sstart(j, u)

      @pl.when(j + LA < NCHUNK)
      def _():
        @pl.when(j + LA >= NB)
        def _():
          swait(j, bn)
        gstart(j + LA, bn)

      scalar_adds(j)
    return 0

  lax.fori_loop(0, NCHUNK // NB, body, 0)

  for u in range(NB):
    swait(NCHUNK - NB + u, u)

  plsc.subcore_barrier()
  for t in range(RPT // ZR):
    @pl.when((s < NS - 1) | (t < 5))
    def _():
      r0 = s * RPT + t * ZR
      pltpu.sync_copy(acc.at[pl.ds(r0, ZR)], agg_out.at[c, pl.ds(r0, ZR)])

  @pl.when(c == 0)
  def _():
    pltpu.sync_copy(sp_acc, wparts_out.at[pl.ds((2 * s) * N, N)])
    pltpu.sync_copy(sm_acc, wparts_out.at[pl.ds((2 * s + 1) * N, N)])


_sc_call = functools.partial(
    pl.kernel,
    out_type=(
        jax.ShapeDtypeStruct((NC, N, HC), jnp.float32),
        jax.ShapeDtypeStruct((NS * 2 * N,), jnp.float32),
    ),
    mesh=plsc.VectorSubcoreMesh(core_axis_name="c", subcore_axis_name="s"),
    compiler_params=pltpu.CompilerParams(
        needs_layout_passes=False, use_tc_tiling_on_sc=False),
    scratch_types=[
        pltpu.VMEM((NCHUNK, K), jnp.int32),
        pltpu.VMEM((NCHUNK, K), jnp.int32),
        pltpu.VMEM((NCHUNK, K), jnp.float32),
        pltpu.VMEM((NB, K, HC), jnp.float32),
        pltpu.VMEM((N,), jnp.float32),
        pltpu.VMEM((N,), jnp.float32),
        pltpu.VMEM_SHARED((N, HC), jnp.float32),
    ] + [pltpu.SemaphoreType.DMA] * (2 * NB),
)(_sc_kernel)


def _tc_kernel(agg_ref, wp_ref, x_ref, m2_ref, selv_ref, a1_ref, c2_ref,
               out_ref):
  pre = jnp.dot(agg_ref[0], m2_ref[:HC], preferred_element_type=jnp.float32)
  pre = pre + jnp.dot(agg_ref[1], m2_ref[HC:],
                      preferred_element_type=jnp.float32)
  pre = pre + jnp.dot(wp_ref[...], selv_ref[...],
                      preferred_element_type=jnp.float32)
  pre = pre + x_ref[...] * a1_ref[...]
  h = jnp.dot(jnp.maximum(pre, 0.0), c2_ref[...],
              preferred_element_type=jnp.float32)
  out_ref[...] = jnp.maximum(h, 0.0)


_BN = 1000
_NP = 2 * NS


def _tc_call(agg, wp, x, m2, selv, a1, c2):
  return pl.pallas_call(
      _tc_kernel,
      grid=(N // _BN,),
      in_specs=[
          pl.BlockSpec((NC, _BN, HC), lambda i: (0, i, 0)),
          pl.BlockSpec((_BN, _NP), lambda i: (i, 0)),
          pl.BlockSpec((_BN, 1), lambda i: (i, 0)),
          pl.BlockSpec((P, P), lambda i: (0, 0)),
          pl.BlockSpec((_NP, P), lambda i: (0, 0)),
          pl.BlockSpec((1, P), lambda i: (0, 0)),
          pl.BlockSpec((P, P), lambda i: (0, 0)),
      ],
      out_specs=pl.BlockSpec((_BN, P), lambda i: (i, 0)),
      out_shape=jax.ShapeDtypeStruct((N, P), jnp.float32),
  )(agg, wp, x, m2, selv, a1, c2)


@jax.jit
def kernel(x, mu, weight, edge_index, theta1_w, theta2_w, theta3_w,
           theta4_w, cat1_w, cat2_w):
  ei = edge_index.astype(jnp.int32)
  e4 = jnp.stack([ei[0] * 2, ei[1]]).reshape(2, NS, NCHUNK, K)
  w3d = weight.reshape(NS, NCHUNK, K)
  mu_rs = mu.reshape(NC * N, HC)

  zeros_hbm = jnp.zeros((N, HC), jnp.float32)
  agg_parts, wparts_flat = _sc_call(e4, w3d, mu_rs, zeros_hbm)
  wp = wparts_flat.reshape(2 * NS, N).T

  a_blk = cat1_w[:P]
  b_blk = cat1_w[P:2 * P]
  c_blk = cat1_w[2 * P:]
  m2 = theta2_w @ b_blk
  a1 = theta1_w @ a_blk
  v3p = jnp.maximum(theta4_w, 0.0) @ theta3_w @ c_blk
  v3m = jnp.maximum(-theta4_w, 0.0) @ theta3_w @ c_blk
  selv = jnp.tile(jnp.concatenate([v3p, v3m], axis=0), (NS, 1))

  return _tc_call(agg_parts, wp, x, m2, selv, a1, cat2_w)


# speedup vs baseline: 1.2186x; 1.2186x over previous
import functools

import jax
import jax.numpy as jnp
from jax import lax
from jax.experimental import pallas as pl
from jax.experimental.pallas import tpu as pltpu
from jax.experimental.pallas import tpu_sc as plsc

N = 10000
E = 320000
P = 128

NC = 2
NS = 16
HC = P // NC
EW = E // NS
K = 16
NCHUNK = EW // K
ZR = 80
RPT = 640


NB = 10
LA = 8


def _sc_kernel(e4, w3d, mu_lo, mu_hi, zeros_hbm, agg_out,
               wparts_out, src_v, dst_v, w_v, rows3, sp_acc, sm_acc, acc,
               *sems):
  gsems = sems[:NB]
  ssems = sems[NB:]
  c = lax.axis_index("c")
  s = lax.axis_index("s")

  z16 = jnp.zeros((16,), jnp.float32)

  @pl.when(c == 0)
  def _():
    def zero_scalar(i, _):
      idx = pl.multiple_of(i * 16, 16)
      sp_acc[pl.ds(idx, 16)] = z16
      sm_acc[pl.ds(idx, 16)] = z16
      return 0
    lax.fori_loop(0, N // 16, zero_scalar, 0)

  for t in range(RPT // ZR):
    @pl.when((s < NS - 1) | (t < 5))
    def _():
      r0 = s * RPT + t * ZR
      pltpu.sync_copy(zeros_hbm.at[pl.ds(r0, ZR)],
                      acc.at[pl.ds(r0, ZR)])
  plsc.subcore_barrier()

  pltpu.sync_copy(e4.at[0, s], src_v)
  pltpu.sync_copy(e4.at[1, s], dst_v)

  @pl.when(c == 0)
  def _():
    pltpu.sync_copy(w3d.at[s], w_v)

  def gstart(j, b):
    @pl.when(c == 0)
    def _():
      pltpu.async_copy(mu_lo.at[src_v.at[j]], rows3.at[b], gsems[b])

    @pl.when(c == 1)
    def _():
      pltpu.async_copy(mu_hi.at[src_v.at[j]], rows3.at[b], gsems[b])

  def gwait(j, b):
    pltpu.make_async_copy(mu_lo.at[src_v.at[j]], rows3.at[b],
                          gsems[b]).wait()

  def sstart(j, b):
    pltpu.async_copy(rows3.at[b], acc.at[dst_v.at[j]], ssems[b], add=True)

  def swait(j, b):
    pltpu.make_async_copy(rows3.at[b], acc.at[dst_v.at[j]],
                          ssems[b]).wait()

  def scalar_adds(j):
    @pl.when(c == 0)
    def _():
      for m in range(K // 16):
        d16 = dst_v[j, pl.ds(m * 16, 16)]
        w16 = w_v[j, pl.ds(m * 16, 16)]
        plsc.addupdate_scatter(sp_acc, [d16], jnp.maximum(w16, 0.0))
        plsc.addupdate_scatter(sm_acc, [d16], jnp.maximum(-w16, 0.0))

  for u in range(LA):
    gstart(u, u)

  def body(jj, _):
    for u in range(NB):
      j = jj * NB + u
      bn = (u + LA) % NB
      gwait(j, u)
      sstart(j, u)

      @pl.when(j + LA < NCHUNK)
      def _():
        @pl.when(j + LA >= NB)
        def _():
          swait(j, bn)
        gstart(j + LA, bn)

      scalar_adds(j)
    return 0

  lax.fori_loop(0, NCHUNK // NB, body, 0)

  for u in range(NB):
    swait(NCHUNK - NB + u, u)

  plsc.subcore_barrier()
  for t in range(RPT // ZR):
    @pl.when((s < NS - 1) | (t < 5))
    def _():
      r0 = s * RPT + t * ZR
      pltpu.sync_copy(acc.at[pl.ds(r0, ZR)], agg_out.at[c, pl.ds(r0, ZR)])

  @pl.when(c == 0)
  def _():
    pltpu.sync_copy(sp_acc, wparts_out.at[pl.ds((2 * s) * N, N)])
    pltpu.sync_copy(sm_acc, wparts_out.at[pl.ds((2 * s + 1) * N, N)])


_sc_call = functools.partial(
    pl.kernel,
    out_type=(
        jax.ShapeDtypeStruct((NC, N, HC), jnp.float32),
        jax.ShapeDtypeStruct((NS * 2 * N,), jnp.float32),
    ),
    mesh=plsc.VectorSubcoreMesh(core_axis_name="c", subcore_axis_name="s"),
    compiler_params=pltpu.CompilerParams(
        needs_layout_passes=False, use_tc_tiling_on_sc=False),
    scratch_types=[
        pltpu.VMEM((NCHUNK, K), jnp.int32),
        pltpu.VMEM((NCHUNK, K), jnp.int32),
        pltpu.VMEM((NCHUNK, K), jnp.float32),
        pltpu.VMEM((NB, K, HC), jnp.float32),
        pltpu.VMEM((N,), jnp.float32),
        pltpu.VMEM((N,), jnp.float32),
        pltpu.VMEM_SHARED((N, HC), jnp.float32),
    ] + [pltpu.SemaphoreType.DMA] * (2 * NB),
)(_sc_kernel)


def _tc_kernel(agg_ref, wp_ref, x_ref, m2_ref, selv_ref, a1_ref, c2_ref,
               out_ref):
  pre = jnp.dot(agg_ref[0], m2_ref[:HC], preferred_element_type=jnp.float32)
  pre = pre + jnp.dot(agg_ref[1], m2_ref[HC:],
                      preferred_element_type=jnp.float32)
  pre = pre + jnp.dot(wp_ref[...], selv_ref[...],
                      preferred_element_type=jnp.float32)
  pre = pre + x_ref[...] * a1_ref[...]
  h = jnp.dot(jnp.maximum(pre, 0.0), c2_ref[...],
              preferred_element_type=jnp.float32)
  out_ref[...] = jnp.maximum(h, 0.0)


_BN = 1000
_NP = 2 * NS


def _tc_call(agg, wp, x, m2, selv, a1, c2):
  return pl.pallas_call(
      _tc_kernel,
      grid=(N // _BN,),
      in_specs=[
          pl.BlockSpec((NC, _BN, HC), lambda i: (0, i, 0)),
          pl.BlockSpec((_BN, _NP), lambda i: (i, 0)),
          pl.BlockSpec((_BN, 1), lambda i: (i, 0)),
          pl.BlockSpec((P, P), lambda i: (0, 0)),
          pl.BlockSpec((_NP, P), lambda i: (0, 0)),
          pl.BlockSpec((1, P), lambda i: (0, 0)),
          pl.BlockSpec((P, P), lambda i: (0, 0)),
      ],
      out_specs=pl.BlockSpec((_BN, P), lambda i: (i, 0)),
      out_shape=jax.ShapeDtypeStruct((N, P), jnp.float32),
  )(agg, wp, x, m2, selv, a1, c2)


@jax.jit
def kernel(x, mu, weight, edge_index, theta1_w, theta2_w, theta3_w,
           theta4_w, cat1_w, cat2_w):
  e4 = edge_index.astype(jnp.int32).reshape(2, NS, NCHUNK, K)
  w3d = weight.reshape(NS, NCHUNK, K)
  mu_lo = mu[:, :HC]
  mu_hi = mu[:, HC:]

  zeros_hbm = jnp.zeros((N, HC), jnp.float32)
  agg_parts, wparts_flat = _sc_call(e4, w3d, mu_lo, mu_hi, zeros_hbm)
  wp = wparts_flat.reshape(2 * NS, N).T

  a_blk = cat1_w[:P]
  b_blk = cat1_w[P:2 * P]
  c_blk = cat1_w[2 * P:]
  m2 = theta2_w @ b_blk
  a1 = theta1_w @ a_blk
  v3p = jnp.maximum(theta4_w, 0.0) @ theta3_w @ c_blk
  v3m = jnp.maximum(-theta4_w, 0.0) @ theta3_w @ c_blk
  selv = jnp.tile(jnp.concatenate([v3p, v3m], axis=0), (NS, 1))

  return _tc_call(agg_parts, wp, x, m2, selv, a1, cat2_w)


# speedup vs baseline: 1.2427x; 1.0198x over previous
import functools

import jax
import jax.numpy as jnp
from jax import lax
from jax.experimental import pallas as pl
from jax.experimental.pallas import tpu as pltpu
from jax.experimental.pallas import tpu_sc as plsc

N = 10000
E = 320000
P = 128

NC = 2
NS = 16
HC = P // NC
EW = E // NS
K = 16
NCHUNK = EW // K
ZR = 80
RPT = 640


NB = 10
LA = 9


def _sc_kernel(e4, w3d, mu_lo, mu_hi, zeros_hbm, agg_out,
               wparts_out, src_v, dst_v, w_v, rows3, sp_acc, sm_acc, acc,
               *sems):
  gsems = sems[:NB]
  ssems = sems[NB:]
  c = lax.axis_index("c")
  s = lax.axis_index("s")

  z16 = jnp.zeros((16,), jnp.float32)

  @pl.when(c == 0)
  def _():
    def zero_scalar(i, _):
      idx = pl.multiple_of(i * 16, 16)
      sp_acc[pl.ds(idx, 16)] = z16
      sm_acc[pl.ds(idx, 16)] = z16
      return 0
    lax.fori_loop(0, N // 16, zero_scalar, 0)

  for t in range(RPT // ZR):
    @pl.when((s < NS - 1) | (t < 5))
    def _():
      r0 = s * RPT + t * ZR
      pltpu.sync_copy(zeros_hbm.at[pl.ds(r0, ZR)],
                      acc.at[pl.ds(r0, ZR)])
  plsc.subcore_barrier()

  pltpu.sync_copy(e4.at[0, s], src_v)
  pltpu.sync_copy(e4.at[1, s], dst_v)

  @pl.when(c == 0)
  def _():
    pltpu.sync_copy(w3d.at[s], w_v)

  def gstart(j, b):
    @pl.when(c == 0)
    def _():
      pltpu.async_copy(mu_lo.at[src_v.at[j]], rows3.at[b], gsems[b])

    @pl.when(c == 1)
    def _():
      pltpu.async_copy(mu_hi.at[src_v.at[j]], rows3.at[b], gsems[b])

  def gwait(j, b):
    pltpu.make_async_copy(mu_lo.at[src_v.at[j]], rows3.at[b],
                          gsems[b]).wait()

  def sstart(j, b):
    pltpu.async_copy(rows3.at[b], acc.at[dst_v.at[j]], ssems[b], add=True)

  def swait(j, b):
    pltpu.make_async_copy(rows3.at[b], acc.at[dst_v.at[j]],
                          ssems[b]).wait()

  def scalar_adds(j):
    @pl.when(c == 0)
    def _():
      for m in range(K // 16):
        d16 = dst_v[j, pl.ds(m * 16, 16)]
        w16 = w_v[j, pl.ds(m * 16, 16)]
        plsc.addupdate_scatter(sp_acc, [d16], jnp.maximum(w16, 0.0))
        plsc.addupdate_scatter(sm_acc, [d16], jnp.maximum(-w16, 0.0))

  for u in range(LA):
    gstart(u, u)

  def body(jj, _):
    for u in range(NB):
      j = jj * NB + u
      bn = (u + LA) % NB
      gwait(j, u)
      sstart(j, u)

      @pl.when(j + LA < NCHUNK)
      def _():
        @pl.when(j + LA >= NB)
        def _():
          swait(j, bn)
        gstart(j + LA, bn)

      scalar_adds(j)
    return 0

  lax.fori_loop(0, NCHUNK // NB, body, 0)

  for u in range(NB):
    swait(NCHUNK - NB + u, u)

  plsc.subcore_barrier()
  for t in range(RPT // ZR):
    @pl.when((s < NS - 1) | (t < 5))
    def _():
      r0 = s * RPT + t * ZR
      pltpu.sync_copy(acc.at[pl.ds(r0, ZR)], agg_out.at[c, pl.ds(r0, ZR)])

  @pl.when(c == 0)
  def _():
    pltpu.sync_copy(sp_acc, wparts_out.at[pl.ds((2 * s) * N, N)])
    pltpu.sync_copy(sm_acc, wparts_out.at[pl.ds((2 * s + 1) * N, N)])


_sc_call = functools.partial(
    pl.kernel,
    out_type=(
        jax.ShapeDtypeStruct((NC, N, HC), jnp.float32),
        jax.ShapeDtypeStruct((NS * 2 * N,), jnp.float32),
    ),
    mesh=plsc.VectorSubcoreMesh(core_axis_name="c", subcore_axis_name="s"),
    compiler_params=pltpu.CompilerParams(
        needs_layout_passes=False, use_tc_tiling_on_sc=False),
    scratch_types=[
        pltpu.VMEM((NCHUNK, K), jnp.int32),
        pltpu.VMEM((NCHUNK, K), jnp.int32),
        pltpu.VMEM((NCHUNK, K), jnp.float32),
        pltpu.VMEM((NB, K, HC), jnp.float32),
        pltpu.VMEM((N,), jnp.float32),
        pltpu.VMEM((N,), jnp.float32),
        pltpu.VMEM_SHARED((N, HC), jnp.float32),
    ] + [pltpu.SemaphoreType.DMA] * (2 * NB),
)(_sc_kernel)


def _tc_kernel(agg_ref, wp_ref, x_ref, m2_ref, selv_ref, a1_ref, c2_ref,
               out_ref):
  pre = jnp.dot(agg_ref[0], m2_ref[:HC], preferred_element_type=jnp.float32)
  pre = pre + jnp.dot(agg_ref[1], m2_ref[HC:],
                      preferred_element_type=jnp.float32)
  pre = pre + jnp.dot(wp_ref[...], selv_ref[...],
                      preferred_element_type=jnp.float32)
  pre = pre + x_ref[...] * a1_ref[...]
  h = jnp.dot(jnp.maximum(pre, 0.0), c2_ref[...],
              preferred_element_type=jnp.float32)
  out_ref[...] = jnp.maximum(h, 0.0)


_BN = 1000
_NP = 2 * NS


def _tc_call(agg, wp, x, m2, selv, a1, c2):
  return pl.pallas_call(
      _tc_kernel,
      grid=(N // _BN,),
      in_specs=[
          pl.BlockSpec((NC, _BN, HC), lambda i: (0, i, 0)),
          pl.BlockSpec((_BN, _NP), lambda i: (i, 0)),
          pl.BlockSpec((_BN, 1), lambda i: (i, 0)),
          pl.BlockSpec((P, P), lambda i: (0, 0)),
          pl.BlockSpec((_NP, P), lambda i: (0, 0)),
          pl.BlockSpec((1, P), lambda i: (0, 0)),
          pl.BlockSpec((P, P), lambda i: (0, 0)),
      ],
      out_specs=pl.BlockSpec((_BN, P), lambda i: (i, 0)),
      out_shape=jax.ShapeDtypeStruct((N, P), jnp.float32),
  )(agg, wp, x, m2, selv, a1, c2)


@jax.jit
def kernel(x, mu, weight, edge_index, theta1_w, theta2_w, theta3_w,
           theta4_w, cat1_w, cat2_w):
  e4 = edge_index.astype(jnp.int32).reshape(2, NS, NCHUNK, K)
  w3d = weight.reshape(NS, NCHUNK, K)
  mu_lo = mu[:, :HC]
  mu_hi = mu[:, HC:]

  zeros_hbm = jnp.zeros((N, HC), jnp.float32)
  agg_parts, wparts_flat = _sc_call(e4, w3d, mu_lo, mu_hi, zeros_hbm)
  wp = wparts_flat.reshape(2 * NS, N).T

  a_blk = cat1_w[:P]
  b_blk = cat1_w[P:2 * P]
  c_blk = cat1_w[2 * P:]
  m2 = theta2_w @ b_blk
  a1 = theta1_w @ a_blk
  v3p = jnp.maximum(theta4_w, 0.0) @ theta3_w @ c_blk
  v3m = jnp.maximum(-theta4_w, 0.0) @ theta3_w @ c_blk
  selv = jnp.tile(jnp.concatenate([v3p, v3m], axis=0), (NS, 1))

  return _tc_call(agg_parts, wp, x, m2, selv, a1, cat2_w)


# speedup vs baseline: 1.2429x; 1.0002x over previous
import functools

import jax
import jax.numpy as jnp
from jax import lax
from jax.experimental import pallas as pl
from jax.experimental.pallas import tpu as pltpu
from jax.experimental.pallas import tpu_sc as plsc

N = 10000
E = 320000
P = 128

NC = 2
NS = 16
HC = P // NC
EW = E // NS
K = 16
NCHUNK = EW // K
ZR = 80
RPT = 640


NB = 10
LA = 9


def _sc_kernel(ei, wflat, mu_lo, mu_hi, zeros_hbm, agg_out,
               wparts_out, src_v, dst_v, w_v, rows3, sp_acc, sm_acc, acc,
               *sems):
  gsems = sems[:NB]
  ssems = sems[NB:]
  c = lax.axis_index("c")
  s = lax.axis_index("s")

  z16 = jnp.zeros((16,), jnp.float32)

  @pl.when(c == 0)
  def _():
    def zero_scalar(i, _):
      idx = pl.multiple_of(i * 16, 16)
      sp_acc[pl.ds(idx, 16)] = z16
      sm_acc[pl.ds(idx, 16)] = z16
      return 0
    lax.fori_loop(0, N // 16, zero_scalar, 0)

  for t in range(RPT // ZR):
    @pl.when((s < NS - 1) | (t < 5))
    def _():
      r0 = s * RPT + t * ZR
      pltpu.sync_copy(zeros_hbm.at[pl.ds(r0, ZR)],
                      acc.at[pl.ds(r0, ZR)])
  plsc.subcore_barrier()

  pltpu.sync_copy(ei.at[0, pl.ds(s * EW, EW)], src_v)
  pltpu.sync_copy(ei.at[1, pl.ds(s * EW, EW)], dst_v)

  @pl.when(c == 0)
  def _():
    pltpu.sync_copy(wflat.at[pl.ds(s * EW, EW)], w_v)

  def eix(j):
    return pl.ds(pl.multiple_of(j * K, 16), K)

  def gstart(j, b):
    @pl.when(c == 0)
    def _():
      pltpu.async_copy(mu_lo.at[src_v.at[eix(j)]], rows3.at[b], gsems[b])

    @pl.when(c == 1)
    def _():
      pltpu.async_copy(mu_hi.at[src_v.at[eix(j)]], rows3.at[b], gsems[b])

  def gwait(j, b):
    pltpu.make_async_copy(mu_lo.at[src_v.at[eix(j)]], rows3.at[b],
                          gsems[b]).wait()

  def sstart(j, b):
    pltpu.async_copy(rows3.at[b], acc.at[dst_v.at[eix(j)]], ssems[b],
                     add=True)

  def swait(j, b):
    pltpu.make_async_copy(rows3.at[b], acc.at[dst_v.at[eix(j)]],
                          ssems[b]).wait()

  def scalar_adds(j):
    @pl.when(c == 0)
    def _():
      for m in range(K // 16):
        d16 = dst_v[pl.ds(pl.multiple_of(j * K + m * 16, 16), 16)]
        w16 = w_v[pl.ds(pl.multiple_of(j * K + m * 16, 16), 16)]
        plsc.addupdate_scatter(sp_acc, [d16], jnp.maximum(w16, 0.0))
        plsc.addupdate_scatter(sm_acc, [d16], jnp.maximum(-w16, 0.0))

  for u in range(LA):
    gstart(u, u)

  def body(jj, _):
    for u in range(NB):
      j = jj * NB + u
      bn = (u + LA) % NB
      gwait(j, u)
      sstart(j, u)

      @pl.when(j + LA < NCHUNK)
      def _():
        @pl.when(j + LA >= NB)
        def _():
          swait(j, bn)
        gstart(j + LA, bn)

      scalar_adds(j)
    return 0

  lax.fori_loop(0, NCHUNK // NB, body, 0)

  for u in range(NB):
    swait(NCHUNK - NB + u, u)

  plsc.subcore_barrier()
  for t in range(RPT // ZR):
    @pl.when((s < NS - 1) | (t < 5))
    def _():
      r0 = s * RPT + t * ZR
      pltpu.sync_copy(acc.at[pl.ds(r0, ZR)], agg_out.at[c, pl.ds(r0, ZR)])

  @pl.when(c == 0)
  def _():
    pltpu.sync_copy(sp_acc, wparts_out.at[pl.ds((2 * s) * N, N)])
    pltpu.sync_copy(sm_acc, wparts_out.at[pl.ds((2 * s + 1) * N, N)])


_sc_call = functools.partial(
    pl.kernel,
    out_type=(
        jax.ShapeDtypeStruct((NC, N, HC), jnp.float32),
        jax.ShapeDtypeStruct((NS * 2 * N,), jnp.float32),
    ),
    mesh=plsc.VectorSubcoreMesh(core_axis_name="c", subcore_axis_name="s"),
    compiler_params=pltpu.CompilerParams(
        needs_layout_passes=False, use_tc_tiling_on_sc=False),
    scratch_types=[
        pltpu.VMEM((EW,), jnp.int32),
        pltpu.VMEM((EW,), jnp.int32),
        pltpu.VMEM((EW,), jnp.float32),
        pltpu.VMEM((NB, K, HC), jnp.float32),
        pltpu.VMEM((N,), jnp.float32),
        pltpu.VMEM((N,), jnp.float32),
        pltpu.VMEM_SHARED((N, HC), jnp.float32),
    ] + [pltpu.SemaphoreType.DMA] * (2 * NB),
)(_sc_kernel)


def _tc_kernel(agg_ref, wp_ref, x_ref, m2_ref, selv_ref, a1_ref, c2_ref,
               out_ref):
  pre = jnp.dot(agg_ref[0], m2_ref[:HC], preferred_element_type=jnp.float32)
  pre = pre + jnp.dot(agg_ref[1], m2_ref[HC:],
                      preferred_element_type=jnp.float32)
  pre = pre + jnp.dot(wp_ref[...], selv_ref[...],
                      preferred_element_type=jnp.float32)
  pre = pre + x_ref[...] * a1_ref[...]
  h = jnp.dot(jnp.maximum(pre, 0.0), c2_ref[...],
              preferred_element_type=jnp.float32)
  out_ref[...] = jnp.maximum(h, 0.0)


_BN = 1000
_NP = 2 * NS


def _tc_call(agg, wp, x, m2, selv, a1, c2):
  return pl.pallas_call(
      _tc_kernel,
      grid=(N // _BN,),
      in_specs=[
          pl.BlockSpec((NC, _BN, HC), lambda i: (0, i, 0)),
          pl.BlockSpec((_BN, _NP), lambda i: (i, 0)),
          pl.BlockSpec((_BN, 1), lambda i: (i, 0)),
          pl.BlockSpec((P, P), lambda i: (0, 0)),
          pl.BlockSpec((_NP, P), lambda i: (0, 0)),
          pl.BlockSpec((1, P), lambda i: (0, 0)),
          pl.BlockSpec((P, P), lambda i: (0, 0)),
      ],
      out_specs=pl.BlockSpec((_BN, P), lambda i: (i, 0)),
      out_shape=jax.ShapeDtypeStruct((N, P), jnp.float32),
  )(agg, wp, x, m2, selv, a1, c2)


@jax.jit
def kernel(x, mu, weight, edge_index, theta1_w, theta2_w, theta3_w,
           theta4_w, cat1_w, cat2_w):
  ei = edge_index.astype(jnp.int32)
  wflat = weight.reshape(E)
  mu_lo = mu[:, :HC]
  mu_hi = mu[:, HC:]

  zeros_hbm = jnp.zeros((N, HC), jnp.float32)
  agg_parts, wparts_flat = _sc_call(ei, wflat, mu_lo, mu_hi, zeros_hbm)
  wp = wparts_flat.reshape(2 * NS, N).T

  a_blk = cat1_w[:P]
  b_blk = cat1_w[P:2 * P]
  c_blk = cat1_w[2 * P:]
  m2 = theta2_w @ b_blk
  a1 = theta1_w @ a_blk
  v3p = jnp.maximum(theta4_w, 0.0) @ theta3_w @ c_blk
  v3m = jnp.maximum(-theta4_w, 0.0) @ theta3_w @ c_blk
  selv = jnp.tile(jnp.concatenate([v3p, v3m], axis=0), (NS, 1))

  return _tc_call(agg_parts, wp, x, m2, selv, a1, cat2_w)
